# baseline (speedup 1.0000x reference)
import functools

import jax
import jax.numpy as jnp
from jax import lax
from jax.experimental import pallas as pl
from jax.experimental.pallas import tpu as pltpu
from jax.experimental.pallas import tpu_sc as plsc

N = 10000
D = 128
E = 320000

NC = 2
NS = 16
NW = NC * NS
CH = 128
NCHUNK = 80
EPT = NCHUNK * CH
E2 = NW * EPT
NPAD = 10112
ZR = NPAD // NS
ZH = 8


def _agg_body(h_hbm, ed_hbm, w_hbm, out_hbm,
              ed0, ed1, dst_s0, dst_s1, rows0, rows1, w_full, zbuf_v, agg_sh,
              gsem0, gsem1, isem0, isem1, ssem0, ssem1):
    cid = lax.axis_index("c")
    sid = lax.axis_index("s")
    wid = cid * NS + sid
    ed = (ed0, ed1)
    dst_s = (dst_s0, dst_s1)
    rows = (rows0, rows1)
    gsem = (gsem0, gsem1)
    isem = (isem0, isem1)
    ssem = (ssem0, ssem1)

    for r in range(ZH):
        for c in range(D // 16):
            zbuf_v[r, pl.ds(c * 16, 16)] = jnp.zeros((16,), jnp.float32)

    def _zcopy(k, _):
        pltpu.sync_copy(zbuf_v, agg_sh.at[pl.ds(sid * ZR + k * ZH, ZH)])
        return 0
    lax.fori_loop(0, ZR // ZH, _zcopy, 0)

    pltpu.sync_copy(w_hbm.at[wid], w_full)

    def fetch_idx(j, b):
        pltpu.async_copy(ed_hbm.at[wid * NCHUNK + j], ed[b], isem[b])

    def wait_idx(b):
        pltpu.make_async_copy(ed_hbm.at[0], ed[b], isem[b]).wait()

    GS = 4

    def start_gather(b):
        for g in range(GS):
            sl = pl.ds(g * (CH // GS), CH // GS)
            pltpu.async_copy(h_hbm.at[ed[b].at[0, sl]], rows[b].at[sl], gsem[b])

    def wait_gather(b):
        for g in range(GS):
            sl = pl.ds(g * (CH // GS), CH // GS)
            pltpu.make_async_copy(h_hbm.at[ed[b].at[0, sl]], rows[b].at[sl],
                                  gsem[b]).wait()

    def start_scatter(b):
        pltpu.async_copy(rows[b], agg_sh.at[dst_s[b]], ssem[b], add=True)

    def wait_scatter(b):
        pltpu.make_async_copy(rows[b], agg_sh.at[dst_s[b]], ssem[b]).wait()

    plsc.subcore_barrier()

    fetch_idx(0, 0)
    wait_idx(0)
    start_gather(0)
    fetch_idx(1, 1)

    def _pair(i, _):
        for b in (0, 1):
            j = 2 * i + b
            o = 1 - b
            wait_gather(b)

            @pl.when(j < NCHUNK - 1)
            def _():
                wait_idx(o)

                @pl.when(j >= 1)
                def _():
                    wait_scatter(o)
                start_gather(o)

            for c in range(CH // 16):
                sl = pl.ds(c * 16, 16)
                dst_s[b][sl] = ed[b][1, sl]

            def _grp(g, _):
                wvec = w_full[j, pl.ds(g * 16, 16)]
                for e16 in range(16):
                    w = wvec[e16]
                    e = g * 16 + e16
                    for r in range(D // 16):
                        sl = pl.ds(r * 16, 16)
                        rows[b][e, sl] = rows[b][e, sl] * w
                return 0
            lax.fori_loop(0, CH // 16, _grp, 0)

            @pl.when(j < NCHUNK - 2)
            def _():
                fetch_idx(j + 2, b)

            start_scatter(b)
        return 0
    lax.fori_loop(0, NCHUNK // 2, _pair, 0)

    wait_scatter(0)
    wait_scatter(1)

    plsc.subcore_barrier()

    pltpu.sync_copy(agg_sh.at[pl.ds(sid * ZR, ZR)],
                    out_hbm.at[cid, pl.ds(sid * ZR, ZR)])


@jax.jit
def _agg(h, edata, wmat):
    mesh = plsc.VectorSubcoreMesh(core_axis_name="c", subcore_axis_name="s")
    return pl.kernel(
        _agg_body,
        out_type=jax.ShapeDtypeStruct((NC, NPAD, D), jnp.float32),
        mesh=mesh,
        scratch_types=[
            pltpu.VMEM((2, CH), jnp.int32),
            pltpu.VMEM((2, CH), jnp.int32),
            pltpu.VMEM((CH,), jnp.int32),
            pltpu.VMEM((CH,), jnp.int32),
            pltpu.VMEM((CH, D), jnp.float32),
            pltpu.VMEM((CH, D), jnp.float32),
            pltpu.VMEM((NCHUNK, CH), jnp.float32),
            pltpu.VMEM((ZH, D), jnp.float32),
            pltpu.VMEM_SHARED((NPAD, D), jnp.float32),
            pltpu.SemaphoreType.DMA,
            pltpu.SemaphoreType.DMA,
            pltpu.SemaphoreType.DMA,
            pltpu.SemaphoreType.DMA,
            pltpu.SemaphoreType.DMA,
            pltpu.SemaphoreType.DMA,
        ],
    )(h, edata, wmat)


def _mlp_body(h_ref, agg_ref, eps_ref, W1_ref, b1_ref, W2_ref, b2_ref,
              g1_ref, B1_ref, g2_ref, B2_ref, out_ref):
    h = h_ref[...]
    agg = agg_ref[0, :N, :] + agg_ref[1, :N, :]
    hp = (1.0 + eps_ref[0, 0]) * h + agg
    y = jnp.dot(hp, W1_ref[...], preferred_element_type=jnp.float32) + b1_ref[...]
    m = jnp.mean(y, axis=0, keepdims=True)
    v = jnp.mean((y - m) ** 2, axis=0, keepdims=True)
    y = g1_ref[...] * (y - m) * lax.rsqrt(v + 1e-5) + B1_ref[...]
    y = jnp.maximum(y, 0.0)
    y = jnp.dot(y, W2_ref[...], preferred_element_type=jnp.float32) + b2_ref[...]
    m = jnp.mean(y, axis=0, keepdims=True)
    v = jnp.mean((y - m) ** 2, axis=0, keepdims=True)
    y = g2_ref[...] * (y - m) * lax.rsqrt(v + 1e-5) + B2_ref[...]
    out_ref[...] = jnp.maximum(y, 0.0)


@jax.jit
def _mlp(h, agg, eps_l, W1, b1, W2, b2, g1, B1, g2, B2):
    vmem = pl.BlockSpec(memory_space=pltpu.VMEM)
    return pl.pallas_call(
        _mlp_body,
        out_shape=jax.ShapeDtypeStruct((N, D), jnp.float32),
        in_specs=[vmem, vmem, pl.BlockSpec(memory_space=pltpu.SMEM)] + [vmem] * 8,
        out_specs=vmem,
    )(h, agg, eps_l, W1, b1, W2, b2, g1, B1, g2, B2)


def kernel(x, edge_index, edge_weight, eps,
           W1_0, b1_0, W2_0, b2_0, bnm_g_0, bnm_b_0, bn_g_0, bn_b_0,
           W1_1, b1_1, W2_1, b2_1, bnm_g_1, bnm_b_1, bn_g_1, bn_b_1):
    pad = E2 - E
    src = jnp.concatenate([edge_index[0], jnp.zeros((pad,), jnp.int32)])
    dst = jnp.concatenate([edge_index[1], jnp.zeros((pad,), jnp.int32)])
    w = jnp.concatenate([edge_weight, jnp.zeros((pad,), jnp.float32)])
    edata = jnp.stack([src.reshape(NW * NCHUNK, CH),
                       dst.reshape(NW * NCHUNK, CH)], axis=1)
    wmat = w.reshape(NW, NCHUNK, CH)
    layers = [
        (W1_0, b1_0, W2_0, b2_0, bnm_g_0, bnm_b_0, bn_g_0, bn_b_0),
        (W1_1, b1_1, W2_1, b2_1, bnm_g_1, bnm_b_1, bn_g_1, bn_b_1),
    ]
    h = x
    for l in range(2):
        W1, b1, W2, b2, g1, B1, g2, B2 = layers[l]
        agg = _agg(h, edata, wmat)
        h = _mlp(h, agg, eps[l].reshape(1, 1),
                 W1, b1.reshape(1, D), W2, b2.reshape(1, D),
                 g1.reshape(1, D), B1.reshape(1, D),
                 g2.reshape(1, D), B2.reshape(1, D))
    return h


# speedup vs baseline: 1.0010x; 1.0010x over previous
import functools

import jax
import jax.numpy as jnp
from jax import lax
from jax.experimental import pallas as pl
from jax.experimental.pallas import tpu as pltpu
from jax.experimental.pallas import tpu_sc as plsc

N = 10000
D = 128
E = 320000

NC = 2
NS = 16
NW = NC * NS
CH = 128
NCHUNK = 80
EPT = NCHUNK * CH
E2 = NW * EPT
NPAD = 10112
ZR = NPAD // NS
ZH = 8


def _agg_body(h_hbm, ed_hbm, w_hbm, out_hbm,
              ed0, ed1, dst_s0, dst_s1, rows0, rows1, w_full, zbuf_v, agg_sh,
              gsem0, gsem1, isem0, isem1, ssem0, ssem1):
    cid = lax.axis_index("c")
    sid = lax.axis_index("s")
    wid = cid * NS + sid
    ed = (ed0, ed1)
    dst_s = (dst_s0, dst_s1)
    rows = (rows0, rows1)
    gsem = (gsem0, gsem1)
    isem = (isem0, isem1)
    ssem = (ssem0, ssem1)

    for r in range(ZH):
        for c in range(D // 16):
            zbuf_v[r, pl.ds(c * 16, 16)] = jnp.zeros((16,), jnp.float32)

    def _zcopy(k, _):
        pltpu.sync_copy(zbuf_v, agg_sh.at[pl.ds(sid * ZR + k * ZH, ZH)])
        return 0
    lax.fori_loop(0, ZR // ZH, _zcopy, 0)

    pltpu.sync_copy(w_hbm.at[wid], w_full)

    def fetch_idx(j, b):
        pltpu.async_copy(ed_hbm.at[wid * NCHUNK + j], ed[b], isem[b])

    def wait_idx(b):
        pltpu.make_async_copy(ed_hbm.at[0], ed[b], isem[b]).wait()

    def start_gather(b):
        pltpu.async_copy(h_hbm.at[ed[b].at[0]], rows[b], gsem[b])

    def wait_gather(b):
        pltpu.make_async_copy(h_hbm.at[ed[b].at[0]], rows[b], gsem[b]).wait()

    def start_scatter(b):
        pltpu.async_copy(rows[b], agg_sh.at[dst_s[b]], ssem[b], add=True)

    def wait_scatter(b):
        pltpu.make_async_copy(rows[b], agg_sh.at[dst_s[b]], ssem[b]).wait()

    plsc.subcore_barrier()

    fetch_idx(0, 0)
    wait_idx(0)
    start_gather(0)
    fetch_idx(1, 1)

    def _pair(i, _):
        for b in (0, 1):
            j = 2 * i + b
            o = 1 - b
            wait_gather(b)

            @pl.when(j < NCHUNK - 1)
            def _():
                wait_idx(o)

                @pl.when(j >= 1)
                def _():
                    wait_scatter(o)
                start_gather(o)

            for c in range(CH // 16):
                sl = pl.ds(c * 16, 16)
                dst_s[b][sl] = ed[b][1, sl]

            def _grp(g, _):
                wvec = w_full[j, pl.ds(g * 16, 16)]
                for e16 in range(16):
                    w = wvec[e16]
                    e = g * 16 + e16
                    for r in range(D // 16):
                        sl = pl.ds(r * 16, 16)
                        rows[b][e, sl] = rows[b][e, sl] * w
                return 0
            lax.fori_loop(0, CH // 16, _grp, 0)

            @pl.when(j < NCHUNK - 2)
            def _():
                fetch_idx(j + 2, b)

            start_scatter(b)
        return 0
    lax.fori_loop(0, NCHUNK // 2, _pair, 0)

    wait_scatter(0)
    wait_scatter(1)

    plsc.subcore_barrier()

    pltpu.sync_copy(agg_sh.at[pl.ds(sid * ZR, ZR)],
                    out_hbm.at[cid, pl.ds(sid * ZR, ZR)])


@jax.jit
def _agg(h, edata, wmat):
    mesh = plsc.VectorSubcoreMesh(core_axis_name="c", subcore_axis_name="s")
    return pl.kernel(
        _agg_body,
        out_type=jax.ShapeDtypeStruct((NC, NPAD, D), jnp.float32),
        mesh=mesh,
        scratch_types=[
            pltpu.VMEM((2, CH), jnp.int32),
            pltpu.VMEM((2, CH), jnp.int32),
            pltpu.VMEM((CH,), jnp.int32),
            pltpu.VMEM((CH,), jnp.int32),
            pltpu.VMEM((CH, D), jnp.float32),
            pltpu.VMEM((CH, D), jnp.float32),
            pltpu.VMEM((NCHUNK, CH), jnp.float32),
            pltpu.VMEM((ZH, D), jnp.float32),
            pltpu.VMEM_SHARED((NPAD, D), jnp.float32),
            pltpu.SemaphoreType.DMA,
            pltpu.SemaphoreType.DMA,
            pltpu.SemaphoreType.DMA,
            pltpu.SemaphoreType.DMA,
            pltpu.SemaphoreType.DMA,
            pltpu.SemaphoreType.DMA,
        ],
    )(h, edata, wmat)


def _mlp_body(h_ref, agg_ref, eps_ref, W1_ref, b1_ref, W2_ref, b2_ref,
              g1_ref, B1_ref, g2_ref, B2_ref, out_ref):
    h = h_ref[...]
    agg = agg_ref[0, :N, :] + agg_ref[1, :N, :]
    hp = (1.0 + eps_ref[0, 0]) * h + agg
    y = jnp.dot(hp, W1_ref[...], preferred_element_type=jnp.float32) + b1_ref[...]
    m = jnp.mean(y, axis=0, keepdims=True)
    v = jnp.mean((y - m) ** 2, axis=0, keepdims=True)
    y = g1_ref[...] * (y - m) * lax.rsqrt(v + 1e-5) + B1_ref[...]
    y = jnp.maximum(y, 0.0)
    y = jnp.dot(y, W2_ref[...], preferred_element_type=jnp.float32) + b2_ref[...]
    m = jnp.mean(y, axis=0, keepdims=True)
    v = jnp.mean((y - m) ** 2, axis=0, keepdims=True)
    y = g2_ref[...] * (y - m) * lax.rsqrt(v + 1e-5) + B2_ref[...]
    out_ref[...] = jnp.maximum(y, 0.0)


@jax.jit
def _mlp(h, agg, eps_l, W1, b1, W2, b2, g1, B1, g2, B2):
    vmem = pl.BlockSpec(memory_space=pltpu.VMEM)
    return pl.pallas_call(
        _mlp_body,
        out_shape=jax.ShapeDtypeStruct((N, D), jnp.float32),
        in_specs=[vmem, vmem, pl.BlockSpec(memory_space=pltpu.SMEM)] + [vmem] * 8,
        out_specs=vmem,
    )(h, agg, eps_l, W1, b1, W2, b2, g1, B1, g2, B2)


def kernel(x, edge_index, edge_weight, eps,
           W1_0, b1_0, W2_0, b2_0, bnm_g_0, bnm_b_0, bn_g_0, bn_b_0,
           W1_1, b1_1, W2_1, b2_1, bnm_g_1, bnm_b_1, bn_g_1, bn_b_1):
    pad = E2 - E
    src = jnp.concatenate([edge_index[0], jnp.zeros((pad,), jnp.int32)])
    dst = jnp.concatenate([edge_index[1], jnp.zeros((pad,), jnp.int32)])
    w = jnp.concatenate([edge_weight, jnp.zeros((pad,), jnp.float32)])
    edata = jnp.stack([src.reshape(NW * NCHUNK, CH),
                       dst.reshape(NW * NCHUNK, CH)], axis=1)
    wmat = w.reshape(NW, NCHUNK, CH)
    layers = [
        (W1_0, b1_0, W2_0, b2_0, bnm_g_0, bnm_b_0, bn_g_0, bn_b_0),
        (W1_1, b1_1, W2_1, b2_1, bnm_g_1, bnm_b_1, bn_g_1, bn_b_1),
    ]
    h = x
    for l in range(2):
        W1, b1, W2, b2, g1, B1, g2, B2 = layers[l]
        agg = _agg(h, edata, wmat)
        h = _mlp(h, agg, eps[l].reshape(1, 1),
                 W1, b1.reshape(1, D), W2, b2.reshape(1, D),
                 g1.reshape(1, D), B1.reshape(1, D),
                 g2.reshape(1, D), B2.reshape(1, D))
    return h
